# baseline (speedup 1.0000x reference)
import jax
import jax.numpy as jnp
from jax import lax
from jax.experimental import pallas as pl
from jax.experimental.pallas import tpu as pltpu

_B, _S, _C = 32, 2048, 1024
_B_BLK = 2
_B_CHUNKS = _B // _B_BLK


def _count_max_hits(x):
    m = jnp.max(x, axis=2, keepdims=True)
    partial = jnp.sum((x == m).astype(jnp.int32), axis=1)
    has_tie = jnp.sum(partial) != _S // 4

    def exact():
        lane = lax.broadcasted_iota(jnp.int32, (1, _S // 4, _C), 2)
        first = jnp.min(jnp.where(x == m, lane, _C), axis=2, keepdims=True)
        onehot = (first == lane).astype(jnp.int32)
        return jnp.sum(onehot, axis=1)

    return jax.lax.cond(has_tie, exact, lambda: partial)


def _precision_kernel(p00_ref, p01_ref, p02_ref, p03_ref, p10_ref, p11_ref,
                      p12_ref, p13_ref, label_ref, out_ref, counts_ref):
    b = pl.program_id(0)

    p0 = (_count_max_hits(p00_ref[...]) + _count_max_hits(p01_ref[...])
          + _count_max_hits(p02_ref[...]) + _count_max_hits(p03_ref[...]))
    p1 = (_count_max_hits(p10_ref[...]) + _count_max_hits(p11_ref[...])
          + _count_max_hits(p12_ref[...]) + _count_max_hits(p13_ref[...]))
    counts_ref[b] = jnp.concatenate([p0, p1], axis=0)

    @pl.when(b == _B_CHUNKS - 1)
    def _():
        counts = counts_ref[...].reshape(_B, _C)
        label = label_ref[...]
        lane2 = lax.broadcasted_iota(jnp.int32, (_B, _C), 1)
        nonzero_cls = lane2 >= 1
        kd = dict(axis=(0, 1), keepdims=True)
        total_char = jnp.sum(jnp.where(nonzero_cls, label, 0), **kd)
        fn = jnp.sum(jnp.where(nonzero_cls, jnp.maximum(label - counts, 0), 0), **kd)
        zero_pred = jnp.sum(jnp.where(lane2 == 0, counts, 0), **kd)
        total_pred = (_B * _S - zero_pred).astype(jnp.float32)
        correct = (total_char - fn).astype(jnp.float32)
        out_ref[...] = correct / (total_pred + 1e-6)


def kernel(pred, label):
    out = pl.pallas_call(
        _precision_kernel,
        grid=(_B_CHUNKS,),
        in_specs=[
            pl.BlockSpec((1, _S // 4, _C), lambda b: (2 * b, 0, 0)),
            pl.BlockSpec((1, _S // 4, _C), lambda b: (2 * b, 1, 0)),
            pl.BlockSpec((1, _S // 4, _C), lambda b: (2 * b, 2, 0)),
            pl.BlockSpec((1, _S // 4, _C), lambda b: (2 * b, 3, 0)),
            pl.BlockSpec((1, _S // 4, _C), lambda b: (2 * b + 1, 0, 0)),
            pl.BlockSpec((1, _S // 4, _C), lambda b: (2 * b + 1, 1, 0)),
            pl.BlockSpec((1, _S // 4, _C), lambda b: (2 * b + 1, 2, 0)),
            pl.BlockSpec((1, _S // 4, _C), lambda b: (2 * b + 1, 3, 0)),
            pl.BlockSpec((_B, _C), lambda b: (0, 0)),
        ],
        out_specs=pl.BlockSpec((1, 1), lambda b: (0, 0)),
        out_shape=jax.ShapeDtypeStruct((1, 1), jnp.float32),
        scratch_shapes=[pltpu.VMEM((_B_CHUNKS, _B_BLK, _C), jnp.int32)],
    )(pred, pred, pred, pred, pred, pred, pred, pred, label)
    return out[0, 0]


# speedup vs baseline: 1.0055x; 1.0055x over previous
import jax
import jax.numpy as jnp
from jax import lax
from jax.experimental import pallas as pl
from jax.experimental.pallas import tpu as pltpu

_B, _S, _C = 32, 2048, 1024
_B_BLK = 2
_B_CHUNKS = _B // _B_BLK


def _count_max_hits(x):
    m = jnp.max(x, axis=2, keepdims=True)
    partial = jnp.sum((x == m).astype(jnp.int32), axis=1)
    has_tie = jnp.sum(partial) != _S // 2

    def exact():
        lane = lax.broadcasted_iota(jnp.int32, (1, _S // 2, _C), 2)
        first = jnp.min(jnp.where(x == m, lane, _C), axis=2, keepdims=True)
        onehot = (first == lane).astype(jnp.int32)
        return jnp.sum(onehot, axis=1)

    return jax.lax.cond(has_tie, exact, lambda: partial)


def _precision_kernel(p00_ref, p01_ref, p10_ref, p11_ref, label_ref, out_ref,
                      counts_ref):
    b = pl.program_id(0)

    p0 = _count_max_hits(p00_ref[...]) + _count_max_hits(p01_ref[...])
    p1 = _count_max_hits(p10_ref[...]) + _count_max_hits(p11_ref[...])
    counts_ref[b] = jnp.concatenate([p0, p1], axis=0)

    @pl.when(b == _B_CHUNKS - 1)
    def _():
        counts = counts_ref[...].reshape(_B, _C)
        label = label_ref[...]
        lane2 = lax.broadcasted_iota(jnp.int32, (_B, _C), 1)
        nonzero_cls = lane2 >= 1
        kd = dict(axis=(0, 1), keepdims=True)
        total_char = jnp.sum(jnp.where(nonzero_cls, label, 0), **kd)
        fn = jnp.sum(jnp.where(nonzero_cls, jnp.maximum(label - counts, 0), 0), **kd)
        zero_pred = jnp.sum(jnp.where(lane2 == 0, counts, 0), **kd)
        total_pred = (_B * _S - zero_pred).astype(jnp.float32)
        correct = (total_char - fn).astype(jnp.float32)
        out_ref[...] = correct / (total_pred + 1e-6)


def kernel(pred, label):
    out = pl.pallas_call(
        _precision_kernel,
        grid=(_B_CHUNKS,),
        in_specs=[
            pl.BlockSpec((1, _S // 2, _C), lambda b: (2 * b, 0, 0)),
            pl.BlockSpec((1, _S // 2, _C), lambda b: (2 * b, 1, 0)),
            pl.BlockSpec((1, _S // 2, _C), lambda b: (2 * b + 1, 0, 0)),
            pl.BlockSpec((1, _S // 2, _C), lambda b: (2 * b + 1, 1, 0)),
            pl.BlockSpec((_B, _C), lambda b: (0, 0)),
        ],
        out_specs=pl.BlockSpec((1, 1), lambda b: (0, 0)),
        out_shape=jax.ShapeDtypeStruct((1, 1), jnp.float32),
        scratch_shapes=[pltpu.VMEM((_B_CHUNKS, _B_BLK, _C), jnp.int32)],
    )(pred, pred, pred, pred, label)
    return out[0, 0]
